# Initial kernel scaffold; baseline (speedup 1.0000x reference)
import jax
import jax.numpy as jnp
from jax.experimental import pallas as pl


def kernel(text, emb_weight, fc_weight, fc_bias):
    raise NotImplementedError("write your pallas kernel here")


# speedup vs baseline: 1.0975x; 1.0975x over previous
import functools

import jax
import jax.numpy as jnp
from jax import lax
from jax.experimental import pallas as pl
from jax.experimental.pallas import tpu as pltpu
from jax.experimental.pallas import tpu_sc as plsc

_B = 16384
_L = 50
_LP = 56
_D = 32
_C = 4
_NC = 2
_NS = 16
_NW = _NC * _NS
_ROWS_W = _B // _NW
_CB = 32
_NCH = _ROWS_W // _CB
_CHUNK_IDX = _CB * _LP


def _embed_bag_sum(text_flat, emb_weight):
    mesh = plsc.VectorSubcoreMesh(core_axis_name="c", subcore_axis_name="s")

    @functools.partial(
        pl.kernel,
        out_type=jax.ShapeDtypeStruct((_B, _D), jnp.float32),
        mesh=mesh,
        scratch_types=[
            pltpu.VMEM((_CHUNK_IDX,), jnp.int32),
            pltpu.VMEM((_CHUNK_IDX,), jnp.int32),
            pltpu.VMEM((_CHUNK_IDX, _D), jnp.float32),
            pltpu.VMEM((_CHUNK_IDX, _D), jnp.float32),
            pltpu.VMEM((_CB, _D), jnp.float32),
            pltpu.VMEM((1, _D), jnp.float32),
            pltpu.SemaphoreType.DMA,
            pltpu.SemaphoreType.DMA,
        ],
        compiler_params=pltpu.CompilerParams(use_tc_tiling_on_sc=False),
    )
    def body(text_hbm, emb_hbm, out_hbm, idx0, idx1, rows0, rows1, outb,
             row0_v, sem0, sem1):
        wid = lax.axis_index("s") * _NC + lax.axis_index("c")
        base_idx = wid * _ROWS_W * _LP
        base_row = wid * _ROWS_W

        pltpu.sync_copy(emb_hbm.at[pl.ds(0, 1)], row0_v)
        pad_n = float(_LP - _L)
        neg_c0 = row0_v[0, pl.ds(0, 16)] * (-pad_n)
        neg_c1 = row0_v[0, pl.ds(16, 16)] * (-pad_n)

        idx_bufs = (idx0, idx1)
        row_bufs = (rows0, rows1)
        sems = (sem0, sem1)

        pltpu.sync_copy(text_hbm.at[pl.ds(base_idx, _CHUNK_IDX)], idx0)
        pending = pltpu.async_copy(emb_hbm.at[idx0], rows0, sem0)

        for ch in range(_NCH):
            cur = ch % 2
            nxt = (ch + 1) % 2
            if ch + 1 < _NCH:
                off = base_idx + (ch + 1) * _CHUNK_IDX
                pltpu.sync_copy(text_hbm.at[pl.ds(off, _CHUNK_IDX)],
                                idx_bufs[nxt])
                nxt_pending = pltpu.async_copy(
                    emb_hbm.at[idx_bufs[nxt]], row_bufs[nxt], sems[nxt])
            pending.wait()
            cur_rows = row_bufs[cur]

            def row_body(i, _):
                def l_body(l, carry):
                    a0, a1 = carry
                    r = i * _LP + l
                    return (a0 + cur_rows[r, pl.ds(0, 16)],
                            a1 + cur_rows[r, pl.ds(16, 16)])
                a0, a1 = lax.fori_loop(0, _LP, l_body, (neg_c0, neg_c1),
                                       unroll=8)
                outb[i, pl.ds(0, 16)] = a0
                outb[i, pl.ds(16, 16)] = a1
                return 0

            lax.fori_loop(0, _CB, row_body, 0)
            pltpu.sync_copy(outb, out_hbm.at[pl.ds(base_row + ch * _CB, _CB)])
            if ch + 1 < _NCH:
                pending = nxt_pending

    return body(text_flat, emb_weight)


def _linear_body(x_ref, w_ref, b_ref, o_ref):
    y = lax.dot_general(x_ref[...], w_ref[...], (((1,), (1,)), ((), ())),
                        preferred_element_type=jnp.float32)
    o_ref[...] = y * (1.0 / _L) + b_ref[...]


def _linear(bag, fc_weight, fc_bias2d):
    return pl.pallas_call(
        _linear_body,
        out_shape=jax.ShapeDtypeStruct((_B, _C), jnp.float32),
    )(bag, fc_weight, fc_bias2d)


def kernel(text, emb_weight, fc_weight, fc_bias):
    text_flat = jnp.pad(text, ((0, 0), (0, _LP - _L))).reshape(-1)
    bag = _embed_bag_sum(text_flat, emb_weight)
    return _linear(bag, fc_weight, fc_bias.reshape(1, _C))


# speedup vs baseline: 2.8975x; 2.6400x over previous
import functools

import jax
import jax.numpy as jnp
from jax import lax
from jax.experimental import pallas as pl
from jax.experimental.pallas import tpu as pltpu
from jax.experimental.pallas import tpu_sc as plsc

_B = 16384
_L = 50
_D = 32
_C = 4
_NC = 2
_NS = 16
_NW = _NC * _NS
_ROWS_W = _B // _NW
_CB = 32
_NCH = _ROWS_W // _CB
_CHUNK_IDX = _CB * _L


def _embed_bag_sum(text_flat, emb_weight):
    mesh = plsc.VectorSubcoreMesh(core_axis_name="c", subcore_axis_name="s")

    @functools.partial(
        pl.kernel,
        out_type=jax.ShapeDtypeStruct((_B, _D), jnp.float32),
        mesh=mesh,
        scratch_types=[
            pltpu.VMEM((_CHUNK_IDX,), jnp.int32),
            pltpu.VMEM((_CHUNK_IDX,), jnp.int32),
            pltpu.VMEM((_CHUNK_IDX, _D), jnp.float32),
            pltpu.VMEM((_CHUNK_IDX, _D), jnp.float32),
            pltpu.VMEM((_CB, _D), jnp.float32),
            pltpu.SemaphoreType.DMA,
            pltpu.SemaphoreType.DMA,
        ],
        compiler_params=pltpu.CompilerParams(use_tc_tiling_on_sc=False),
    )
    def body(text_hbm, emb_hbm, out_hbm, idx0, idx1, rows0, rows1, outb,
             sem0, sem1):
        wid = lax.axis_index("s") * _NC + lax.axis_index("c")
        base_idx = wid * _ROWS_W * _L
        base_row = wid * _ROWS_W

        idx_bufs = (idx0, idx1)
        row_bufs = (rows0, rows1)
        sems = (sem0, sem1)

        zero = jnp.zeros((16,), jnp.float32)

        pltpu.sync_copy(text_hbm.at[pl.ds(base_idx, _CHUNK_IDX)], idx0)
        pending = pltpu.async_copy(emb_hbm.at[idx0], rows0, sem0)

        for ch in range(_NCH):
            cur = ch % 2
            nxt = (ch + 1) % 2
            if ch + 1 < _NCH:
                off = base_idx + (ch + 1) * _CHUNK_IDX
                pltpu.sync_copy(text_hbm.at[pl.ds(off, _CHUNK_IDX)],
                                idx_bufs[nxt])
                nxt_pending = pltpu.async_copy(
                    emb_hbm.at[idx_bufs[nxt]], row_bufs[nxt], sems[nxt])
            pending.wait()
            cur_rows = row_bufs[cur]

            def row_body(i, _):
                def l_body(l, carry):
                    a0, a1 = carry
                    r = i * _L + l
                    return (a0 + cur_rows[r, pl.ds(0, 16)],
                            a1 + cur_rows[r, pl.ds(16, 16)])
                a0, a1 = lax.fori_loop(0, _L, l_body, (zero, zero),
                                       unroll=10)
                outb[i, pl.ds(0, 16)] = a0
                outb[i, pl.ds(16, 16)] = a1
                return 0

            lax.fori_loop(0, _CB, row_body, 0)
            pltpu.sync_copy(outb, out_hbm.at[pl.ds(base_row + ch * _CB, _CB)])
            if ch + 1 < _NCH:
                pending = nxt_pending

    return body(text_flat, emb_weight)


def _linear_body(x_ref, w_ref, b_ref, o_ref):
    y = lax.dot_general(x_ref[...], w_ref[...], (((1,), (1,)), ((), ())),
                        preferred_element_type=jnp.float32)
    o_ref[...] = y * (1.0 / _L) + b_ref[...]


def _linear(bag, fc_weight, fc_bias2d):
    return pl.pallas_call(
        _linear_body,
        out_shape=jax.ShapeDtypeStruct((_B, _C), jnp.float32),
    )(bag, fc_weight, fc_bias2d)


def kernel(text, emb_weight, fc_weight, fc_bias):
    bag = _embed_bag_sum(text.reshape(-1), emb_weight)
    return _linear(bag, fc_weight, fc_bias.reshape(1, _C))
